# Initial kernel scaffold; baseline (speedup 1.0000x reference)
import jax
import jax.numpy as jnp
from jax.experimental import pallas as pl


def kernel(x, edge_index, W1, b1, W2, b2, W3, b3, Wa1, ba1, Wa2, ba2, W_ih, b_ih, W_hh, b_hh, Wo1, bo1, Wo2, bo2, Wout, bout):
    raise NotImplementedError("write your pallas kernel here")


# speedup vs baseline: 14.8535x; 14.8535x over previous
import functools

import jax
import jax.numpy as jnp
from jax import lax
from jax.experimental import pallas as pl
from jax.experimental.pallas import tpu as pltpu
from jax.experimental.pallas import tpu_sc as plsc

N = 10000
T = 12
H = 64
E = 640000
FC = 6
DT = 6.0 / 5.0

NC = 2
NS = 16
NW = NC * NS
BATCH = 128
KJ = 157
EPW = KJ * BATCH
EPAD = NW * EPW
NPAD = 10112
RPT = NPAD // NS
DPT = 624

BN = 2000
GN = N // BN
BA = 400

_HI = lax.Precision.DEFAULT


def _mm(a, b):
    return lax.dot_general(a, b, (((1,), (0,)), ((), ())),
                           precision=_HI, preferred_element_type=jnp.float32)


def _mmT(a, b):
    return lax.dot_general(a, b, (((1,), (1,)), ((), ())),
                           precision=_HI, preferred_element_type=jnp.float32)


@functools.cache
def _make_spmm(D):
    mesh = plsc.VectorSubcoreMesh(core_axis_name="c", subcore_axis_name="s")

    @functools.partial(
        pl.kernel,
        out_type=jax.ShapeDtypeStruct((NC, N, D), jnp.float32),
        mesh=mesh,
        compiler_params=pltpu.CompilerParams(use_tc_tiling_on_sc=False),
        scratch_types=[
            pltpu.VMEM((KJ, BATCH), jnp.int32),
            pltpu.VMEM((KJ, BATCH), jnp.int32),
            pltpu.VMEM((BATCH, D), jnp.float32),
            pltpu.VMEM_SHARED((NPAD, D), jnp.float32),
            pltpu.SemaphoreType.DMA,
        ],
    )
    def spmm(table, src3, dst3, zrows, out, src_v, dst_v, rows, acc, sem):
        c = lax.axis_index("c")
        s = lax.axis_index("s")
        wid = c * NS + s
        pltpu.sync_copy(zrows, acc.at[pl.ds(s * RPT, RPT)])
        pltpu.sync_copy(src3.at[wid], src_v)
        pltpu.sync_copy(dst3.at[wid], dst_v)
        plsc.subcore_barrier()

        def body(j, carry):
            pltpu.async_copy(table.at[src_v.at[j]], rows, sem).wait()
            pltpu.sync_copy(rows, acc.at[dst_v.at[j]], add=True)
            return carry

        lax.fori_loop(0, KJ, body, 0)
        plsc.subcore_barrier()

        @pl.when(s < NS - 1)
        def _():
            pltpu.sync_copy(acc.at[pl.ds(s * DPT, DPT)],
                            out.at[c, pl.ds(s * DPT, DPT)])

        @pl.when(s == NS - 1)
        def _():
            pltpu.sync_copy(acc.at[pl.ds((NS - 1) * DPT, N - (NS - 1) * DPT)],
                            out.at[c, pl.ds((NS - 1) * DPT, N - (NS - 1) * DPT)])

    return spmm


def _prep_body(p_ref, xp_ref, dinv_ref, g1_ref):
    deg = p_ref[0, :, 0:1] + p_ref[1, :, 0:1] + 1.0
    dinv = lax.rsqrt(deg)
    dinv_ref[...] = jnp.broadcast_to(dinv, dinv_ref.shape)
    g1_ref[...] = dinv * xp_ref[...]


def _tc_prep(p, xp):
    return pl.pallas_call(
        _prep_body,
        grid=(GN,),
        in_specs=[
            pl.BlockSpec((2, BN, 16), lambda i: (0, i, 0)),
            pl.BlockSpec((BN, 16), lambda i: (i, 0)),
        ],
        out_specs=[
            pl.BlockSpec((BN, 16), lambda i: (i, 0)),
            pl.BlockSpec((BN, 16), lambda i: (i, 0)),
        ],
        out_shape=[
            jax.ShapeDtypeStruct((N, 16), jnp.float32),
            jax.ShapeDtypeStruct((N, 16), jnp.float32),
        ],
    )(p, xp)


def _post1_body(s_ref, g1_ref, dinv_ref, a1_ref):
    dv = dinv_ref[:, 0:1]
    a1_ref[...] = dv * (s_ref[0] + s_ref[1] + g1_ref[...])


def _tc_post1(s, g1, dinv):
    return pl.pallas_call(
        _post1_body,
        grid=(GN,),
        in_specs=[
            pl.BlockSpec((2, BN, 16), lambda i: (0, i, 0)),
            pl.BlockSpec((BN, 16), lambda i: (i, 0)),
            pl.BlockSpec((BN, 16), lambda i: (i, 0)),
        ],
        out_specs=pl.BlockSpec((BN, 16), lambda i: (i, 0)),
        out_shape=jax.ShapeDtypeStruct((N, 16), jnp.float32),
    )(s, g1, dinv)


def _layer1_body(a1_ref, dinv_ref, w1_ref, b1_ref, w2_ref, g2_ref):
    t = pl.program_id(0)
    cols = lax.broadcasted_iota(jnp.int32, (1, 16), 1)
    a_col = jnp.sum(jnp.where(cols == t, a1_ref[...], 0.0),
                    axis=1, keepdims=True)
    h1 = jnp.maximum(a_col * w1_ref[...] + b1_ref[...], 0.0)
    g2_ref[0] = dinv_ref[:, 0:1] * _mm(h1, w2_ref[...])


def _tc_layer1(a1, dinv, w1, b1, w2):
    return pl.pallas_call(
        _layer1_body,
        grid=(T, GN),
        in_specs=[
            pl.BlockSpec((BN, 16), lambda t, i: (i, 0)),
            pl.BlockSpec((BN, 16), lambda t, i: (i, 0)),
            pl.BlockSpec((1, H), lambda t, i: (0, 0)),
            pl.BlockSpec((1, H), lambda t, i: (0, 0)),
            pl.BlockSpec((H, H), lambda t, i: (0, 0)),
        ],
        out_specs=pl.BlockSpec((1, BN, H), lambda t, i: (t, i, 0)),
        out_shape=jax.ShapeDtypeStruct((T, N, H), jnp.float32),
    )(a1, dinv, w1, b1, w2)


def _l23_body(s_ref, g_ref, dinv_ref, b_ref, wn_ref, out_ref):
    dv = dinv_ref[:, 0:1]
    a = dv * (s_ref[0, 0] + s_ref[0, 1] + g_ref[0]) + b_ref[...]
    h = jnp.maximum(a, 0.0)
    out_ref[0] = dv * _mm(h, wn_ref[...])


def _tc_l23(s, g, dinv, b, wn):
    return pl.pallas_call(
        _l23_body,
        grid=(T, GN),
        in_specs=[
            pl.BlockSpec((1, 2, BN, H), lambda t, i: (t, 0, i, 0)),
            pl.BlockSpec((1, BN, H), lambda t, i: (t, i, 0)),
            pl.BlockSpec((BN, 16), lambda t, i: (i, 0)),
            pl.BlockSpec((1, H), lambda t, i: (0, 0)),
            pl.BlockSpec((H, H), lambda t, i: (0, 0)),
        ],
        out_specs=pl.BlockSpec((1, BN, H), lambda t, i: (t, i, 0)),
        out_shape=jax.ShapeDtypeStruct((T, N, H), jnp.float32),
    )(s, g, dinv, b, wn)


def _att_body(s_ref, g_ref, dinv_ref, b3_ref, wa1_ref, ba1_ref, wa2_ref,
              ba2_ref, wr_ref, wz_ref, wn_ref, br_ref, bz_ref, bn_ref,
              hr_ref, hz_ref, hn_ref, wo1_ref, y_ref, gn_ref):
    dv = dinv_ref[:, 0:1]
    ssum = s_ref[:, 0] + s_ref[:, 1] + g_ref[...]
    h3 = jnp.maximum(dv[None] * ssum + b3_ref[...], 0.0)
    flat = h3.reshape(T * BA, H)
    t1 = jnp.tanh(_mm(flat, wa1_ref[...]) + ba1_ref[...])
    att = jnp.sum(t1 * wa2_ref[...], axis=1, keepdims=True) + ba2_ref[0, 0]
    att = att.reshape(T, BA, 1)
    m = jnp.max(att, axis=0, keepdims=True)
    e = jnp.exp(att - m)
    w = e / jnp.sum(e, axis=0, keepdims=True)
    nf = jnp.sum(h3 * w, axis=0)
    r = jax.nn.sigmoid(_mmT(nf, wr_ref[...]) + br_ref[...] + hr_ref[...])
    z = jax.nn.sigmoid(_mmT(nf, wz_ref[...]) + bz_ref[...] + hz_ref[...])
    cand = jnp.tanh(_mmT(nf, wn_ref[...]) + bn_ref[...] + r * hn_ref[...])
    y = (1.0 - z) * cand
    y_ref[...] = y
    gn_ref[...] = dv * _mm(y, wo1_ref[...])


def _tc_att(s, g, dinv, b3, wa1, ba1, wa2, ba2, wr, wz, wn, br, bz, bn,
            hr, hz, hn, wo1):
    full = lambda shape: pl.BlockSpec(shape, lambda i: tuple(0 for _ in shape))
    return pl.pallas_call(
        _att_body,
        grid=(N // BA,),
        in_specs=[
            pl.BlockSpec((T, 2, BA, H), lambda i: (0, 0, i, 0)),
            pl.BlockSpec((T, BA, H), lambda i: (0, i, 0)),
            pl.BlockSpec((BA, 16), lambda i: (i, 0)),
            full((1, H)), full((H, H)), full((1, H)), full((1, H)),
            full((1, 1)),
            full((H, H)), full((H, H)), full((H, H)),
            full((1, H)), full((1, H)), full((1, H)),
            full((1, H)), full((1, H)), full((1, H)),
            full((H, H)),
        ],
        out_specs=[
            pl.BlockSpec((BA, H), lambda i: (i, 0)),
            pl.BlockSpec((BA, H), lambda i: (i, 0)),
        ],
        out_shape=[
            jax.ShapeDtypeStruct((N, H), jnp.float32),
            jax.ShapeDtypeStruct((N, H), jnp.float32),
        ],
    )(s, g, dinv, b3, wa1, ba1, wa2, ba2, wr, wz, wn, br, bz, bn, hr, hz, hn,
      wo1)


def _fmid_body(s_ref, g_ref, dinv_ref, bo1_ref, wo2_ref, out_ref):
    dv = dinv_ref[:, 0:1]
    t1 = jnp.tanh(dv * (s_ref[0] + s_ref[1] + g_ref[...]) + bo1_ref[...])
    out_ref[...] = dv * _mm(t1, wo2_ref[...])


def _tc_fmid(s, g, dinv, bo1, wo2):
    return pl.pallas_call(
        _fmid_body,
        grid=(GN,),
        in_specs=[
            pl.BlockSpec((2, BN, H), lambda i: (0, i, 0)),
            pl.BlockSpec((BN, H), lambda i: (i, 0)),
            pl.BlockSpec((BN, 16), lambda i: (i, 0)),
            pl.BlockSpec((1, H), lambda i: (0, 0)),
            pl.BlockSpec((H, H), lambda i: (0, 0)),
        ],
        out_specs=pl.BlockSpec((BN, H), lambda i: (i, 0)),
        out_shape=jax.ShapeDtypeStruct((N, H), jnp.float32),
    )(s, g, dinv, bo1, wo2)


@functools.cache
def _make_fend_mid(coef):
    def body(s_ref, g_ref, dinv_ref, bo2_ref, y_ref, wo1_ref, k_ref, gn_ref):
        dv = dinv_ref[:, 0:1]
        k = jnp.tanh(dv * (s_ref[0] + s_ref[1] + g_ref[...]) + bo2_ref[...])
        k_ref[...] = k
        yn = y_ref[...] + coef * k
        gn_ref[...] = dv * _mm(yn, wo1_ref[...])

    def call(s, g, dinv, bo2, y, wo1):
        return pl.pallas_call(
            body,
            grid=(GN,),
            in_specs=[
                pl.BlockSpec((2, BN, H), lambda i: (0, i, 0)),
                pl.BlockSpec((BN, H), lambda i: (i, 0)),
                pl.BlockSpec((BN, 16), lambda i: (i, 0)),
                pl.BlockSpec((1, H), lambda i: (0, 0)),
                pl.BlockSpec((BN, H), lambda i: (i, 0)),
                pl.BlockSpec((H, H), lambda i: (0, 0)),
            ],
            out_specs=[
                pl.BlockSpec((BN, H), lambda i: (i, 0)),
                pl.BlockSpec((BN, H), lambda i: (i, 0)),
            ],
            out_shape=[
                jax.ShapeDtypeStruct((N, H), jnp.float32),
                jax.ShapeDtypeStruct((N, H), jnp.float32),
            ],
        )(s, g, dinv, bo2, y, wo1)

    return call


def _fend_last_body(s_ref, g_ref, dinv_ref, bo2_ref, y_ref, k1_ref, k2_ref,
                    k3_ref, wo1_ref, yn_ref, gn_ref):
    dv = dinv_ref[:, 0:1]
    k4 = jnp.tanh(dv * (s_ref[0] + s_ref[1] + g_ref[...]) + bo2_ref[...])
    yn = y_ref[...] + (DT / 6.0) * (k1_ref[...] + 2.0 * k2_ref[...]
                                    + 2.0 * k3_ref[...] + k4)
    yn_ref[...] = yn
    gn_ref[...] = dv * _mm(yn, wo1_ref[...])


def _tc_fend_last(s, g, dinv, bo2, y, k1, k2, k3, wo1):
    return pl.pallas_call(
        _fend_last_body,
        grid=(GN,),
        in_specs=[
            pl.BlockSpec((2, BN, H), lambda i: (0, i, 0)),
            pl.BlockSpec((BN, H), lambda i: (i, 0)),
            pl.BlockSpec((BN, 16), lambda i: (i, 0)),
            pl.BlockSpec((1, H), lambda i: (0, 0)),
            pl.BlockSpec((BN, H), lambda i: (i, 0)),
            pl.BlockSpec((BN, H), lambda i: (i, 0)),
            pl.BlockSpec((BN, H), lambda i: (i, 0)),
            pl.BlockSpec((BN, H), lambda i: (i, 0)),
            pl.BlockSpec((H, H), lambda i: (0, 0)),
        ],
        out_specs=[
            pl.BlockSpec((BN, H), lambda i: (i, 0)),
            pl.BlockSpec((BN, H), lambda i: (i, 0)),
        ],
        out_shape=[
            jax.ShapeDtypeStruct((N, H), jnp.float32),
            jax.ShapeDtypeStruct((N, H), jnp.float32),
        ],
    )(s, g, dinv, bo2, y, k1, k2, k3, wo1)


def _pred_body(ys_ref, wout_ref, bout_ref, out_ref):
    p = jnp.sum(ys_ref[...] * wout_ref[...][None], axis=2) + bout_ref[0, 0]
    out_ref[...] = p


def _tc_pred(ys, wout, bout):
    return pl.pallas_call(
        _pred_body,
        in_specs=[
            pl.BlockSpec((FC, N, H), lambda: (0, 0, 0)),
            pl.BlockSpec((1, H), lambda: (0, 0)),
            pl.BlockSpec((1, 1), lambda: (0, 0)),
        ],
        out_specs=pl.BlockSpec((FC, N), lambda: (0, 0)),
        out_shape=jax.ShapeDtypeStruct((FC, N), jnp.float32),
    )(ys, wout, bout)


def kernel(x, edge_index, W1, b1, W2, b2, W3, b3, Wa1, ba1, Wa2, ba2,
           W_ih, b_ih, W_hh, b_hh, Wo1, bo1, Wo2, bo2, Wout, bout):
    f32 = jnp.float32
    src = edge_index[0].astype(jnp.int32)
    dst = edge_index[1].astype(jnp.int32)
    pad = EPAD - E
    src3 = jnp.concatenate([src, jnp.zeros((pad,), jnp.int32)]).reshape(
        NW, KJ, BATCH)
    dst3 = jnp.concatenate([dst, jnp.full((pad,), N, jnp.int32)]).reshape(
        NW, KJ, BATCH)
    z16 = jnp.zeros((RPT, 16), f32)
    z64 = jnp.zeros((RPT, H), f32)
    ones16 = jnp.ones((N, 16), f32)
    xp = jnp.pad(x[0], ((0, 0), (0, 16 - T)))

    spmm16 = _make_spmm(16)
    spmm64 = _make_spmm(H)

    b1r, b2r, b3r = b1[None], b2[None], b3[None]
    ba1r, ba2r = ba1[None], jnp.reshape(ba2, (1, 1))
    bo1r, bo2r = bo1[None], bo2[None]
    woutr, boutr = Wout.reshape(1, H), jnp.reshape(bout, (1, 1))
    wr, wz, wn = W_ih[:H], W_ih[H:2 * H], W_ih[2 * H:]
    br, bz, bn = b_ih[None, :H], b_ih[None, H:2 * H], b_ih[None, 2 * H:]
    hr, hz, hn = b_hh[None, :H], b_hh[None, H:2 * H], b_hh[None, 2 * H:]

    degp = spmm16(ones16, src3, dst3, z16)
    dinv, g1 = _tc_prep(degp, xp)

    s1 = spmm16(g1, src3, dst3, z16)
    a1 = _tc_post1(s1, g1, dinv)
    g2 = _tc_layer1(a1, dinv, W1, b1r, W2)
    s2 = jnp.stack([spmm64(g2[t], src3, dst3, z64) for t in range(T)])
    g3 = _tc_l23(s2, g2, dinv, b2r, W3)
    s3 = jnp.stack([spmm64(g3[t], src3, dst3, z64) for t in range(T)])

    y, g = _tc_att(s3, g3, dinv, b3r, Wa1, ba1r, Wa2.reshape(1, H), ba2r,
                   wr, wz, wn, br, bz, bn, hr, hz, hn, Wo1)

    fmid1 = _make_fend_mid(DT * 0.5)
    fmid2 = _make_fend_mid(DT)
    ys = [y]
    for _ in range(FC - 1):
        ks = []
        for coef_fn in (fmid1, fmid1, fmid2):
            sa = spmm64(g, src3, dst3, z64)
            gm = _tc_fmid(sa, g, dinv, bo1r, Wo2)
            sb = spmm64(gm, src3, dst3, z64)
            k, g = coef_fn(sb, gm, dinv, bo2r, y, Wo1)
            ks.append(k)
        sa = spmm64(g, src3, dst3, z64)
        gm = _tc_fmid(sa, g, dinv, bo1r, Wo2)
        sb = spmm64(gm, src3, dst3, z64)
        y, g = _tc_fend_last(sb, gm, dinv, bo2r, y, ks[0], ks[1], ks[2], Wo1)
        ys.append(y)

    pred = _tc_pred(jnp.stack(ys), woutr, boutr)
    return jnp.transpose(pred)[None]
